# Initial kernel scaffold; baseline (speedup 1.0000x reference)
import jax
import jax.numpy as jnp
from jax.experimental import pallas as pl


def kernel(x, global_pe, week_pe, month_pe, year_pe):
    raise NotImplementedError("write your pallas kernel here")


# speedup vs baseline: 2.5721x; 2.5721x over previous
import functools

import jax
import jax.numpy as jnp
from jax.experimental import pallas as pl
from jax.experimental.pallas import tpu as pltpu

_BS = 1024


def _body(g_ref, w_ref, m_ref, y_ref, x_ref, o_ref, pe_ref):
    b = pl.program_id(1)

    @pl.when(b == 0)
    def _compute_pe():
        i = pl.program_id(0)
        bs = x_ref.shape[1]
        pos = jax.lax.broadcasted_iota(jnp.int32, (bs, 1), 0) + i * bs
        pe_ref[:, 0 : g_ref.shape[1]] = g_ref[...]
        off = g_ref.shape[1]
        for t_ref in (w_ref, m_ref, y_ref):
            p, dp = t_ref.shape
            lanes = jax.lax.broadcasted_iota(jnp.int32, (bs, p), 1)
            onehot = (lanes == pos % p).astype(jnp.float32)
            pe_ref[:, off : off + dp] = jnp.dot(
                onehot, t_ref[...], preferred_element_type=jnp.float32
            )
            off += dp

    o_ref[...] = x_ref[...] + pe_ref[...][None]


@jax.jit
def kernel(x, global_pe, week_pe, month_pe, year_pe):
    B, S, D = x.shape
    d_g = global_pe.shape[1]
    grid = (S // _BS, B)
    return pl.pallas_call(
        _body,
        grid=grid,
        in_specs=[
            pl.BlockSpec((_BS, d_g), lambda i, b: (i, 0)),
            pl.BlockSpec(week_pe.shape, lambda i, b: (0, 0)),
            pl.BlockSpec(month_pe.shape, lambda i, b: (0, 0)),
            pl.BlockSpec(year_pe.shape, lambda i, b: (0, 0)),
            pl.BlockSpec((1, _BS, D), lambda i, b: (b, i, 0)),
        ],
        out_specs=pl.BlockSpec((1, _BS, D), lambda i, b: (b, i, 0)),
        out_shape=jax.ShapeDtypeStruct(x.shape, x.dtype),
        scratch_shapes=[pltpu.VMEM((_BS, D), jnp.float32)],
        compiler_params=pltpu.CompilerParams(
            dimension_semantics=("arbitrary", "arbitrary"),
        ),
    )(global_pe, week_pe, month_pe, year_pe, x)
